# baseline (speedup 1.0000x reference)
import functools

import jax
import jax.numpy as jnp
from jax.experimental import pallas as pl
from jax.experimental.pallas import tpu as pltpu
from jax.experimental.pallas import tpu_sc as plsc

NPOINT = 512
RADIUS = 0.4
NSAMPLE = 32
B = 4
N = 8192
RBLK = 128
TD = 16
TROW = 128
TOT = NSAMPLE * B * NPOINT
TCHUNK = 4096
GCH = 512


def _fps_kernel(far0_ref, xyz_ref, newx_ref):
    x = xyz_ref[:, 0, :]
    y = xyz_ref[:, 1, :]
    z = xyz_ref[:, 2, :]
    far = far0_ref[...]
    iota_n = jax.lax.broadcasted_iota(jnp.int32, (B, N), 1)
    iota_p = jax.lax.broadcasted_iota(jnp.int32, (B, NPOINT), 1)

    dist0 = jnp.full((B, N), 1e10, jnp.float32)
    cxs0 = jnp.zeros((B, NPOINT), jnp.float32)
    cys0 = jnp.zeros((B, NPOINT), jnp.float32)
    czs0 = jnp.zeros((B, NPOINT), jnp.float32)

    def body(i, st):
        dist, far, cxs, cys, czs = st
        sel = iota_p == i
        oh = (iota_n == far).astype(jnp.float32)
        cx = jnp.sum(x * oh, axis=1, keepdims=True)
        cy = jnp.sum(y * oh, axis=1, keepdims=True)
        cz = jnp.sum(z * oh, axis=1, keepdims=True)
        cxs = jnp.where(sel, cx, cxs)
        cys = jnp.where(sel, cy, cys)
        czs = jnp.where(sel, cz, czs)
        d = (x - cx) ** 2 + (y - cy) ** 2 + (z - cz) ** 2
        dist = jnp.minimum(dist, d)
        m = jnp.max(dist, axis=1, keepdims=True)
        cand = jnp.where(dist == m, iota_n, N)
        far = jnp.min(cand, axis=1, keepdims=True)
        return (dist, far, cxs, cys, czs)

    _, _, cxs, cys, czs = jax.lax.fori_loop(
        0, NPOINT, body, (dist0, far, cxs0, cys0, czs0))
    newx_ref[:, 0, :] = cxs
    newx_ref[:, 1, :] = cys
    newx_ref[:, 2, :] = czs


def _ballq_idx_kernel(xyzt_ref, newx_ref, outi_ref):
    X = xyzt_ref[0]
    CT = newx_ref[0]
    xn = jnp.sum(X * X, axis=1, keepdims=True)
    cn = jnp.sum(CT * CT, axis=0, keepdims=True)
    D = (-2.0 * jnp.dot(X, CT, preferred_element_type=jnp.float32)
         + cn) + xn
    mask = jnp.logical_not(D > RADIUS * RADIUS)
    maskf = mask.astype(jnp.float32)

    pos = maskf
    sh = 1
    while sh < N:
        pos = pos + jnp.concatenate(
            [jnp.zeros((sh, RBLK), jnp.float32), pos[: N - sh, :]], axis=0)
        sh *= 2
    cnt = pos[N - 1 :, :]
    V = jnp.where(mask, pos, 0.0)

    iota_row = jax.lax.broadcasted_iota(
        jnp.int32, (1, N), 1).astype(jnp.float32)
    boff = (pl.program_id(0) * N).astype(jnp.float32)

    g0 = None
    for s in range(NSAMPLE):
        ind = jnp.where(V == float(s + 1), 1.0, 0.0)
        g = jnp.dot(iota_row, ind,
                    preferred_element_type=jnp.float32)
        if s == 0:
            g0 = g
        else:
            g = jnp.where(cnt > float(s), g, g0)
        outi_ref[s : s + 1, :] = (g + boff).astype(jnp.int32)


def _transpose_kernel(x_ref, out_ref):
    out_ref[...] = jnp.transpose(x_ref[:, 0:TD], (1, 0))


def _mlp_kernel(x_ref, cmat_ref, w0_ref, b0_ref, g0_ref, be0_ref,
                w1_ref, b1_ref, g1_ref, be1_ref,
                w2_ref, b2_ref, g2_ref, be2_ref, out_ref):
    csub = jnp.concatenate([cmat_ref[...]] * NSAMPLE, axis=1)
    h = jnp.concatenate(
        [x_ref[0:3, :] - csub, x_ref[3:6, :]], axis=0)
    for wr, br, gr, ber in ((w0_ref, b0_ref, g0_ref, be0_ref),
                            (w1_ref, b1_ref, g1_ref, be1_ref),
                            (w2_ref, b2_ref, g2_ref, be2_ref)):
        h = jnp.dot(wr[...], h, preferred_element_type=jnp.float32) + br[...]
        mean = jnp.mean(h, axis=1, keepdims=True)
        var = jnp.mean((h - mean) ** 2, axis=1, keepdims=True)
        h = (h - mean) / jnp.sqrt(var + 1e-5)
        h = gr[...] * h + ber[...]
        h = jnp.maximum(h, 0.0)
    m2 = B * NPOINT
    acc = h[:, 0:m2]
    for s in range(1, NSAMPLE):
        acc = jnp.maximum(acc, h[:, s * m2 : (s + 1) * m2])
    for b in range(B):
        out_ref[b, :, :] = acc[:, b * NPOINT : (b + 1) * NPOINT]


def kernel(xyz, points, W0, b0, gamma0, beta0, W1, b1, gamma1, beta1,
           W2, b2, gamma2, beta2):
    far0 = jax.random.randint(
        jax.random.key(42), (B,), 0, N, dtype=jnp.int32).reshape(B, 1)

    new_xyz = pl.pallas_call(
        _fps_kernel,
        out_shape=jax.ShapeDtypeStruct((B, 3, NPOINT), jnp.float32),
    )(far0, xyz)

    xyz_t = jnp.transpose(xyz, (0, 2, 1))
    points_t = jnp.transpose(points, (0, 2, 1))
    nblk = NPOINT // RBLK
    idx2d = pl.pallas_call(
        _ballq_idx_kernel,
        grid=(B, nblk),
        in_specs=[
            pl.BlockSpec((1, N, 3), lambda b, r: (b, 0, 0)),
            pl.BlockSpec((1, 3, RBLK), lambda b, r: (b, 0, r)),
        ],
        out_specs=pl.BlockSpec((NSAMPLE, RBLK),
                               lambda b, r: (0, b * (NPOINT // RBLK) + r)),
        out_shape=jax.ShapeDtypeStruct((NSAMPLE, B * NPOINT), jnp.int32),
    )(xyz_t, new_xyz)
    flat_idx = idx2d.reshape(TOT)

    table = jnp.concatenate(
        [xyz_t, points_t, jnp.zeros((B, N, TROW - 6), jnp.float32)],
        axis=2).reshape(B * N, TROW)

    info = plsc.get_sparse_core_info()
    nw = info.num_cores * info.num_subcores
    bpw = TOT // nw
    nch = bpw // GCH
    mesh = plsc.VectorSubcoreMesh(core_axis_name="c", subcore_axis_name="s")

    @functools.partial(
        pl.kernel, mesh=mesh,
        out_type=jax.ShapeDtypeStruct((TOT, TROW), jnp.float32),
        scratch_types=[
            pltpu.VMEM((GCH,), jnp.int32),
            pltpu.VMEM((GCH, TROW), jnp.float32),
            pltpu.SemaphoreType.DMA,
        ],
    )
    def _sc_gather(table_hbm, idx_hbm, out_hbm, idx_v, rows_v, sem):
        wid = jax.lax.axis_index("s") * info.num_cores + jax.lax.axis_index("c")
        base = wid * bpw
        for c in range(nch):
            off = base + c * GCH
            pltpu.sync_copy(idx_hbm.at[pl.ds(off, GCH)], idx_v)
            pltpu.async_copy(table_hbm.at[idx_v], rows_v, sem).wait()
            pltpu.sync_copy(rows_v, out_hbm.at[pl.ds(off, GCH)])

    gathered = _sc_gather(table, flat_idx)

    x_in = pl.pallas_call(
        _transpose_kernel,
        grid=(TOT // TCHUNK,),
        in_specs=[pl.BlockSpec((TCHUNK, TROW), lambda i: (i, 0))],
        out_specs=pl.BlockSpec((TD, TCHUNK), lambda i: (0, i)),
        out_shape=jax.ShapeDtypeStruct((TD, TOT), jnp.float32),
    )(gathered)

    cmat = jnp.transpose(new_xyz, (1, 0, 2)).reshape(3, B * NPOINT)

    new_points = pl.pallas_call(
        _mlp_kernel,
        out_shape=jax.ShapeDtypeStruct((B, 64, NPOINT), jnp.float32),
    )(x_in, cmat,
      W0, b0.reshape(-1, 1), gamma0.reshape(-1, 1), beta0.reshape(-1, 1),
      W1, b1.reshape(-1, 1), gamma1.reshape(-1, 1), beta1.reshape(-1, 1),
      W2, b2.reshape(-1, 1), gamma2.reshape(-1, 1), beta2.reshape(-1, 1))

    return (new_xyz, new_points)


# speedup vs baseline: 1.0078x; 1.0078x over previous
import functools

import jax
import jax.numpy as jnp
from jax.experimental import pallas as pl
from jax.experimental.pallas import tpu as pltpu
from jax.experimental.pallas import tpu_sc as plsc

NPOINT = 512
RADIUS = 0.4
NSAMPLE = 32
B = 4
N = 8192
RBLK = 128
TD = 16
TROW = 128
TOT = NSAMPLE * B * NPOINT
TCHUNK = 4096
GCH = 512


def _fps_kernel(far0_ref, xyz_ref, newx_ref):
    x = xyz_ref[:, 0, :]
    y = xyz_ref[:, 1, :]
    z = xyz_ref[:, 2, :]
    far = far0_ref[...]
    iota_n = jax.lax.broadcasted_iota(jnp.int32, (B, N), 1)
    iota_p = jax.lax.broadcasted_iota(jnp.int32, (B, NPOINT), 1)

    dist0 = jnp.full((B, N), 1e10, jnp.float32)
    cxs0 = jnp.zeros((B, NPOINT), jnp.float32)
    cys0 = jnp.zeros((B, NPOINT), jnp.float32)
    czs0 = jnp.zeros((B, NPOINT), jnp.float32)

    def body(i, st):
        dist, far, cxs, cys, czs = st
        sel = iota_p == i
        oh = (iota_n == far).astype(jnp.float32)
        cx = jnp.sum(x * oh, axis=1, keepdims=True)
        cy = jnp.sum(y * oh, axis=1, keepdims=True)
        cz = jnp.sum(z * oh, axis=1, keepdims=True)
        cxs = jnp.where(sel, cx, cxs)
        cys = jnp.where(sel, cy, cys)
        czs = jnp.where(sel, cz, czs)
        d = (x - cx) ** 2 + (y - cy) ** 2 + (z - cz) ** 2
        dist = jnp.minimum(dist, d)
        m = jnp.max(dist, axis=1, keepdims=True)
        cand = jnp.where(dist == m, iota_n, N)
        far = jnp.min(cand, axis=1, keepdims=True)
        return (dist, far, cxs, cys, czs)

    _, _, cxs, cys, czs = jax.lax.fori_loop(
        0, NPOINT, body, (dist0, far, cxs0, cys0, czs0))
    newx_ref[:, 0, :] = cxs
    newx_ref[:, 1, :] = cys
    newx_ref[:, 2, :] = czs


def _ballq_idx_kernel(xyzt_ref, newx_ref, outi_ref):
    X = xyzt_ref[0]
    CT = newx_ref[0]
    xn = jnp.sum(X * X, axis=1, keepdims=True)
    cn = jnp.sum(CT * CT, axis=0, keepdims=True)
    D = (-2.0 * jnp.dot(X, CT, preferred_element_type=jnp.float32)
         + cn) + xn
    mask = jnp.logical_not(D > RADIUS * RADIUS)
    maskf = mask.astype(jnp.float32)

    pos = maskf
    sh = 1
    while sh < N:
        pos = pos + jnp.concatenate(
            [jnp.zeros((sh, RBLK), jnp.float32), pos[: N - sh, :]], axis=0)
        sh *= 2
    cnt = pos[N - 1 :, :]
    V = jnp.where(mask, pos, 0.0)

    iota_row = jax.lax.broadcasted_iota(
        jnp.int32, (1, N), 1).astype(jnp.float32)
    boff = (pl.program_id(0) * N).astype(jnp.float32)

    g0 = None
    for s in range(NSAMPLE):
        ind = jnp.where(V == float(s + 1), 1.0, 0.0)
        g = jnp.dot(iota_row, ind,
                    preferred_element_type=jnp.float32)
        if s == 0:
            g0 = g
        else:
            g = jnp.where(cnt > float(s), g, g0)
        outi_ref[s : s + 1, :] = (g + boff).astype(jnp.int32)


def _transpose_kernel(x_ref, out_ref):
    out_ref[...] = jnp.transpose(x_ref[:, 0:TD], (1, 0))


def _mlp_kernel(x_ref, x2_ref, cmat_ref, w0_ref, b0_ref, g0_ref, be0_ref,
                w1_ref, b1_ref, g1_ref, be1_ref,
                w2_ref, b2_ref, g2_ref, be2_ref, out_ref):
    xa = jnp.concatenate([x_ref[0:6, :], x2_ref[0:6, :]], axis=1)
    csub = jnp.concatenate([cmat_ref[...]] * NSAMPLE, axis=1)
    h = jnp.concatenate(
        [xa[0:3, :] - csub, xa[3:6, :]], axis=0)
    for wr, br, gr, ber in ((w0_ref, b0_ref, g0_ref, be0_ref),
                            (w1_ref, b1_ref, g1_ref, be1_ref),
                            (w2_ref, b2_ref, g2_ref, be2_ref)):
        h = jnp.dot(wr[...], h, preferred_element_type=jnp.float32) + br[...]
        mean = jnp.mean(h, axis=1, keepdims=True)
        var = jnp.mean((h - mean) ** 2, axis=1, keepdims=True)
        h = (h - mean) / jnp.sqrt(var + 1e-5)
        h = gr[...] * h + ber[...]
        h = jnp.maximum(h, 0.0)
    m2 = B * NPOINT
    acc = h[:, 0:m2]
    for s in range(1, NSAMPLE):
        acc = jnp.maximum(acc, h[:, s * m2 : (s + 1) * m2])
    for b in range(B):
        out_ref[b, :, :] = acc[:, b * NPOINT : (b + 1) * NPOINT]


def kernel(xyz, points, W0, b0, gamma0, beta0, W1, b1, gamma1, beta1,
           W2, b2, gamma2, beta2):
    far0 = jax.random.randint(
        jax.random.key(42), (B,), 0, N, dtype=jnp.int32).reshape(B, 1)

    new_xyz = pl.pallas_call(
        _fps_kernel,
        out_shape=jax.ShapeDtypeStruct((B, 3, NPOINT), jnp.float32),
    )(far0, xyz)

    xyz_t = jnp.transpose(xyz, (0, 2, 1))
    points_t = jnp.transpose(points, (0, 2, 1))
    nblk = NPOINT // RBLK
    idx2d = pl.pallas_call(
        _ballq_idx_kernel,
        grid=(B, nblk),
        in_specs=[
            pl.BlockSpec((1, N, 3), lambda b, r: (b, 0, 0)),
            pl.BlockSpec((1, 3, RBLK), lambda b, r: (b, 0, r)),
        ],
        out_specs=pl.BlockSpec((NSAMPLE, RBLK),
                               lambda b, r: (0, b * (NPOINT // RBLK) + r)),
        out_shape=jax.ShapeDtypeStruct((NSAMPLE, B * NPOINT), jnp.int32),
    )(xyz_t, new_xyz)
    flat_idx = idx2d.reshape(TOT)

    table = jnp.concatenate(
        [xyz_t, points_t, jnp.zeros((B, N, TROW - 6), jnp.float32)],
        axis=2).reshape(B * N, TROW)

    info = plsc.get_sparse_core_info()
    nw = info.num_cores * info.num_subcores
    half = TOT // 2
    bpw = half // nw
    nch = bpw // GCH
    mesh = plsc.VectorSubcoreMesh(core_axis_name="c", subcore_axis_name="s")

    @functools.partial(
        pl.kernel, mesh=mesh,
        out_type=jax.ShapeDtypeStruct((half, TROW), jnp.float32),
        scratch_types=[
            pltpu.VMEM((GCH,), jnp.int32),
            pltpu.VMEM((GCH, TROW), jnp.float32),
            pltpu.SemaphoreType.DMA,
        ],
    )
    def _sc_gather(table_hbm, idx_hbm, out_hbm, idx_v, rows_v, sem):
        wid = jax.lax.axis_index("s") * info.num_cores + jax.lax.axis_index("c")
        base = wid * bpw
        for c in range(nch):
            off = base + c * GCH
            pltpu.sync_copy(idx_hbm.at[pl.ds(off, GCH)], idx_v)
            pltpu.async_copy(table_hbm.at[idx_v], rows_v, sem).wait()
            pltpu.sync_copy(rows_v, out_hbm.at[pl.ds(off, GCH)])

    def _transpose_half(g):
        return pl.pallas_call(
            _transpose_kernel,
            grid=(half // TCHUNK,),
            in_specs=[pl.BlockSpec((TCHUNK, TROW), lambda i: (i, 0))],
            out_specs=pl.BlockSpec((TD, TCHUNK), lambda i: (0, i)),
            out_shape=jax.ShapeDtypeStruct((TD, half), jnp.float32),
        )(g)

    gathered1 = _sc_gather(table, flat_idx[:half])
    gathered2 = _sc_gather(table, flat_idx[half:])
    x_in1 = _transpose_half(gathered1)
    x_in2 = _transpose_half(gathered2)

    cmat = jnp.transpose(new_xyz, (1, 0, 2)).reshape(3, B * NPOINT)

    new_points = pl.pallas_call(
        _mlp_kernel,
        out_shape=jax.ShapeDtypeStruct((B, 64, NPOINT), jnp.float32),
    )(x_in1, x_in2, cmat,
      W0, b0.reshape(-1, 1), gamma0.reshape(-1, 1), beta0.reshape(-1, 1),
      W1, b1.reshape(-1, 1), gamma1.reshape(-1, 1), beta1.reshape(-1, 1),
      W2, b2.reshape(-1, 1), gamma2.reshape(-1, 1), beta2.reshape(-1, 1))

    return (new_xyz, new_points)


# speedup vs baseline: 1.1595x; 1.1505x over previous
import jax
import jax.numpy as jnp
from jax.experimental import pallas as pl

NPOINT = 512
RADIUS = 0.4
NSAMPLE = 32
B = 4
N = 8192
RBLK = 128
NCH = 8


def _fps_kernel(far0_ref, xyz_ref, newx_ref):
    x = xyz_ref[:, 0, :]
    y = xyz_ref[:, 1, :]
    z = xyz_ref[:, 2, :]
    far = far0_ref[...]
    iota_n = jax.lax.broadcasted_iota(jnp.int32, (B, N), 1)
    iota_p = jax.lax.broadcasted_iota(jnp.int32, (B, NPOINT), 1)

    dist0 = jnp.full((B, N), 1e10, jnp.float32)
    cxs0 = jnp.zeros((B, NPOINT), jnp.float32)
    cys0 = jnp.zeros((B, NPOINT), jnp.float32)
    czs0 = jnp.zeros((B, NPOINT), jnp.float32)

    def body(i, st):
        dist, far, cxs, cys, czs = st
        sel = iota_p == i
        oh = (iota_n == far).astype(jnp.float32)
        cx = jnp.sum(x * oh, axis=1, keepdims=True)
        cy = jnp.sum(y * oh, axis=1, keepdims=True)
        cz = jnp.sum(z * oh, axis=1, keepdims=True)
        cxs = jnp.where(sel, cx, cxs)
        cys = jnp.where(sel, cy, cys)
        czs = jnp.where(sel, cz, czs)
        d = (x - cx) ** 2 + (y - cy) ** 2 + (z - cz) ** 2
        dist = jnp.minimum(dist, d)
        m = jnp.max(dist, axis=1, keepdims=True)
        cand = jnp.where(dist == m, iota_n, N)
        far = jnp.min(cand, axis=1, keepdims=True)
        return (dist, far, cxs, cys, czs)

    _, _, cxs, cys, czs = jax.lax.fori_loop(
        0, NPOINT, body, (dist0, far, cxs0, cys0, czs0))
    newx_ref[:, 0, :] = cxs
    newx_ref[:, 1, :] = cys
    newx_ref[:, 2, :] = czs


def _ballq_kernel(xyzt_ref, xyz3_ref, pts3_ref, newx_ref, out_ref):
    X = xyzt_ref[0]
    CT = newx_ref[0]
    xn = jnp.sum(X * X, axis=1, keepdims=True)
    cn = jnp.sum(CT * CT, axis=0, keepdims=True)
    D = (-2.0 * jnp.dot(X, CT, preferred_element_type=jnp.float32)
         + cn) + xn
    mask = jnp.logical_not(D > RADIUS * RADIUS)
    maskf = mask.astype(jnp.float32)

    pos = maskf
    sh = 1
    while sh < N:
        pos = pos + jnp.concatenate(
            [jnp.zeros((sh, RBLK), jnp.float32), pos[: N - sh, :]], axis=0)
        sh *= 2
    cnt = pos[N - 1 :, :]
    V = jnp.where(mask, pos, 0.0)

    iota_row = jax.lax.broadcasted_iota(
        jnp.int32, (1, N), 1).astype(jnp.float32)
    FT = jnp.concatenate(
        [xyz3_ref[0], pts3_ref[0], iota_row,
         jnp.zeros((1, N), jnp.float32)], axis=0)
    cpad = jnp.concatenate(
        [CT, jnp.zeros((NCH - 3, RBLK), jnp.float32)], axis=0)

    g0 = None
    for s in range(NSAMPLE):
        ind = jnp.where(V == float(s + 1), 1.0, 0.0)
        g = jnp.dot(FT, ind, preferred_element_type=jnp.float32)
        if s == 0:
            g0 = g
        else:
            g = jnp.where(cnt > float(s), g, g0)
        out_ref[:, s, :] = g - cpad


def _mlp_kernel(x_ref, w0_ref, b0_ref, g0_ref, be0_ref,
                w1_ref, b1_ref, g1_ref, be1_ref,
                w2_ref, b2_ref, g2_ref, be2_ref, out_ref):
    h = x_ref[0:6, :]
    for wr, br, gr, ber in ((w0_ref, b0_ref, g0_ref, be0_ref),
                            (w1_ref, b1_ref, g1_ref, be1_ref),
                            (w2_ref, b2_ref, g2_ref, be2_ref)):
        h = jnp.dot(wr[...], h, preferred_element_type=jnp.float32) + br[...]
        mean = jnp.mean(h, axis=1, keepdims=True)
        var = jnp.mean((h - mean) ** 2, axis=1, keepdims=True)
        h = (h - mean) / jnp.sqrt(var + 1e-5)
        h = gr[...] * h + ber[...]
        h = jnp.maximum(h, 0.0)
    m2 = B * NPOINT
    acc = h[:, 0:m2]
    for s in range(1, NSAMPLE):
        acc = jnp.maximum(acc, h[:, s * m2 : (s + 1) * m2])
    for b in range(B):
        out_ref[b, :, :] = acc[:, b * NPOINT : (b + 1) * NPOINT]


def kernel(xyz, points, W0, b0, gamma0, beta0, W1, b1, gamma1, beta1,
           W2, b2, gamma2, beta2):
    far0 = jax.random.randint(
        jax.random.key(42), (B,), 0, N, dtype=jnp.int32).reshape(B, 1)

    new_xyz = pl.pallas_call(
        _fps_kernel,
        out_shape=jax.ShapeDtypeStruct((B, 3, NPOINT), jnp.float32),
    )(far0, xyz)

    xyz_t = jnp.transpose(xyz, (0, 2, 1))
    nblk = NPOINT // RBLK
    grouped = pl.pallas_call(
        _ballq_kernel,
        grid=(B, nblk),
        in_specs=[
            pl.BlockSpec((1, N, 3), lambda b, r: (b, 0, 0)),
            pl.BlockSpec((1, 3, N), lambda b, r: (b, 0, 0)),
            pl.BlockSpec((1, 3, N), lambda b, r: (b, 0, 0)),
            pl.BlockSpec((1, 3, RBLK), lambda b, r: (b, 0, r)),
        ],
        out_specs=pl.BlockSpec((NCH, NSAMPLE, RBLK),
                               lambda b, r: (0, 0, b * (NPOINT // RBLK) + r)),
        out_shape=jax.ShapeDtypeStruct((NCH, NSAMPLE, B * NPOINT),
                                       jnp.float32),
    )(xyz_t, xyz, points, new_xyz)

    x_in = grouped.reshape(NCH, NSAMPLE * B * NPOINT)

    new_points = pl.pallas_call(
        _mlp_kernel,
        out_shape=jax.ShapeDtypeStruct((B, 64, NPOINT), jnp.float32),
    )(x_in,
      W0, b0.reshape(-1, 1), gamma0.reshape(-1, 1), beta0.reshape(-1, 1),
      W1, b1.reshape(-1, 1), gamma1.reshape(-1, 1), beta1.reshape(-1, 1),
      W2, b2.reshape(-1, 1), gamma2.reshape(-1, 1), beta2.reshape(-1, 1))

    return (new_xyz, new_points)
